# baseline (speedup 1.0000x reference)
import functools

import jax
import jax.numpy as jnp
from jax import lax
from jax.experimental import pallas as pl
from jax.experimental.pallas import tpu as pltpu
from jax.experimental.pallas import tpu_sc as plsc

L = 16
NC = 2
NS = 16
NW = NC * NS
DIM = 64
CHUNK = 256
GROUPS = DIM // L


def _sc_body(total, nums_hbm, types_hbm, ntab_hbm, ttab_hbm, out_hbm,
             idx_v, tidx_v, nrows_v, trows_v, sem_n, sem_t):
    c = lax.axis_index("c")
    s = lax.axis_index("s")
    wid = s * NC + c
    per_w = total // NW
    n_chunks = per_w // CHUNK
    base_w = wid * per_w

    def chunk_body(ci, carry):
        base = base_w + ci * CHUNK
        pltpu.sync_copy(nums_hbm.at[pl.ds(base, CHUNK)], idx_v)
        pltpu.sync_copy(types_hbm.at[pl.ds(base, CHUNK)], tidx_v)
        cp_n = pltpu.async_copy(ntab_hbm.at[idx_v], nrows_v, sem_n)
        cp_t = pltpu.async_copy(ttab_hbm.at[tidx_v], trows_v, sem_t)
        cp_n.wait()
        cp_t.wait()

        def row_body(r, rcarry):
            for g in range(GROUPS):
                sl = pl.ds(g * L, L)
                nrows_v[r, sl] = nrows_v[r, sl] + trows_v[r, sl]
            return rcarry

        lax.fori_loop(0, CHUNK, row_body, 0)
        pltpu.sync_copy(nrows_v, out_hbm.at[pl.ds(base, CHUNK)])
        return carry

    lax.fori_loop(0, n_chunks, chunk_body, 0)


def kernel(task_nums, task_types, task_num_table, task_type_table):
    B, T = task_nums.shape
    total = B * T
    nums = task_nums.reshape(total).astype(jnp.int32)
    types = task_types.reshape(total).astype(jnp.int32)

    mesh = plsc.VectorSubcoreMesh(core_axis_name="c", subcore_axis_name="s")
    call = pl.kernel(
        functools.partial(_sc_body, total),
        out_type=jax.ShapeDtypeStruct((total, DIM), jnp.float32),
        mesh=mesh,
        scratch_types=[
            pltpu.VMEM((CHUNK,), jnp.int32),
            pltpu.VMEM((CHUNK,), jnp.int32),
            pltpu.VMEM((CHUNK, DIM), jnp.float32),
            pltpu.VMEM((CHUNK, DIM), jnp.float32),
            pltpu.SemaphoreType.DMA,
            pltpu.SemaphoreType.DMA,
        ],
        compiler_params=pltpu.CompilerParams(use_tc_tiling_on_sc=False),
    )
    out = call(nums, types, task_num_table, task_type_table)
    return out.reshape(B, T, DIM)
